# baseline (speedup 1.0000x reference)
import functools

import jax
import jax.numpy as jnp
from jax import lax
from jax.experimental import pallas as pl
from jax.experimental.pallas import tpu as pltpu
from jax.experimental.pallas import tpu_sc as plsc

_N_SC = 2
_N_SUB = 16
_CHUNK = 128
_HALF = 32
_GRPS = _CHUNK // 16


_UNROLL = 6
_RING = 3


def _layer_body(n_total, nch, ep, ibp, wp, zrows, out, acc, ib, wb, rows,
                contrib, *sems):
    c = lax.axis_index("c")
    s = lax.axis_index("s")
    rows_per_sub = n_total // _N_SUB
    roff = pl.multiple_of(s * rows_per_sub, 8)
    gsem = sems[0:3]
    ssem = sems[3:6]
    isem = sems[6:12]

    pltpu.sync_copy(zrows, acc.at[pl.ds(roff, rows_per_sub)])
    plsc.subcore_barrier()

    ep_c = ep.at[c]
    tile_base = s * nch

    def stage(j, slot):
        pltpu.async_copy(ibp.at[tile_base + j], ib.at[slot], isem[slot])
        pltpu.async_copy(wp.at[tile_base + j], wb.at[slot], isem[slot])

    def stage_wait(slot):
        pltpu.make_async_copy(ibp.at[tile_base], ib.at[slot],
                              isem[slot]).wait()
        pltpu.make_async_copy(wp.at[tile_base], wb.at[slot],
                              isem[slot]).wait()

    def gather(slot, b):
        pltpu.async_copy(ep_c.at[ib.at[slot, 0]], rows.at[b], gsem[b])

    def gather_wait(b):
        pltpu.make_async_copy(ep_c.at[pl.ds(0, _CHUNK)], rows.at[b],
                              gsem[b]).wait()

    def scat(slot, b):
        pltpu.async_copy(contrib.at[b], acc.at[ib.at[slot, 1]], ssem[b],
                         add=True)

    def scat_wait(b):
        pltpu.make_async_copy(contrib.at[b], acc.at[pl.ds(0, _CHUNK)],
                              ssem[b]).wait()

    def compute(b, slot):
        def group_body(g, carry):
            gb = pl.multiple_of(g * 16, 16)
            wgrp = wb[slot, pl.ds(gb, 16)]
            for e2 in range(16):
                e = gb + e2
                wv = wgrp[e2]
                va, vb2 = plsc.unpack(rows[b, e],
                                      format=plsc.PackFormat.INTERLEAVED)
                contrib[b, e, pl.ds(0, 16)] = va * wv
                contrib[b, e, pl.ds(16, 16)] = vb2 * wv
            return carry

        lax.fori_loop(0, _GRPS, group_body, 0)

    ni = nch // _UNROLL

    for j0 in range(3):
        stage(j0, j0)
    for j0 in range(2):
        stage_wait(j0)
        gather(j0, j0)

    def iter_body(i, carry):
        for k in range(_UNROLL):
            b = k % _RING
            j = _UNROLL * i + k
            gather_wait(b)
            if k >= 3:
                scat_wait(b)
            else:
                @pl.when(i > 0)
                def _():
                    scat_wait(b)
            if k < 3:
                stage(j + 3, (k + 3) % 6)
            else:
                @pl.when(i < ni - 1)
                def _():
                    stage(j + 3, (k + 3) % 6)
            if k < 4:
                stage_wait((k + 2) % 6)
                gather((k + 2) % 6, (k + 2) % _RING)
            else:
                @pl.when(i < ni - 1)
                def _():
                    stage_wait((k + 2) % 6)
                    gather((k + 2) % 6, (k + 2) % _RING)
            compute(b, k)
            scat(k, b)
        return carry

    lax.fori_loop(0, ni, iter_body, 0)
    for b in range(_RING):
        scat_wait(b)

    plsc.subcore_barrier()
    pltpu.sync_copy(acc.at[pl.ds(roff, rows_per_sub)],
                    out.at[c].at[pl.ds(roff, rows_per_sub)])


def _make_layer(n_total, nch):
    mesh = plsc.VectorSubcoreMesh(core_axis_name="c", subcore_axis_name="s")
    return pl.kernel(
        functools.partial(_layer_body, n_total, nch),
        out_type=jax.ShapeDtypeStruct((_N_SC, n_total, _HALF), jnp.float32),
        mesh=mesh,
        scratch_types=[
            pltpu.VMEM_SHARED((n_total, _HALF), jnp.float32),
            pltpu.VMEM((6, 2, _CHUNK), jnp.int32),
            pltpu.VMEM((6, _CHUNK), jnp.float32),
            pltpu.VMEM((_RING, _CHUNK, _HALF), jnp.bfloat16),
            pltpu.VMEM((_RING, _CHUNK, _HALF), jnp.float32),
        ] + [pltpu.SemaphoreType.DMA] * 12,
        compiler_params=pltpu.CompilerParams(use_tc_tiling_on_sc=False,
                                             needs_layout_passes=False),
    )


def kernel(user_emb, item_emb, edge_weight, edge_index):
    n_users, dim = user_emb.shape
    n_total = n_users + item_emb.shape[0]
    n_edges = edge_weight.shape[0]
    n_layers = 3

    row_grain = _N_SUB * 8
    n_tpad = ((n_total + row_grain - 1) // row_grain) * row_grain

    grain = _N_SUB * _CHUNK * _UNROLL
    n_pad = ((n_edges + grain - 1) // grain) * grain
    pad = n_pad - n_edges
    nch = n_pad // (_N_SUB * _CHUNK)
    src = jnp.concatenate(
        [edge_index[1].astype(jnp.int32), jnp.zeros((pad,), jnp.int32)])
    dst = jnp.concatenate(
        [edge_index[0].astype(jnp.int32), jnp.zeros((pad,), jnp.int32)])
    w = jnp.concatenate(
        [edge_weight.astype(jnp.float32), jnp.zeros((pad,), jnp.float32)])
    ibp = jnp.stack(
        [src.reshape(-1, _CHUNK), dst.reshape(-1, _CHUNK)], axis=1)
    wp = w.reshape(-1, _CHUNK)

    def to_bf16_interleaved(x):
        return jnp.stack([x[..., :16], x[..., 16:]], axis=-1) \
            .reshape(*x.shape[:-1], 32).astype(jnp.bfloat16)

    e0 = jnp.concatenate([user_emb, item_emb], axis=0) \
        .reshape(n_total, _N_SC, _HALF).transpose(1, 0, 2)
    ep = jnp.zeros((_N_SC, n_tpad, _HALF), jnp.float32).at[:, :n_total].set(e0)
    zrows = jnp.zeros((n_tpad // _N_SUB, _HALF), jnp.float32)

    layer = _make_layer(n_tpad, nch)
    for _ in range(n_layers):
        ep = layer(to_bf16_interleaved(ep), ibp, wp, zrows)

    e_full = ep[:, :n_total].transpose(1, 0, 2).reshape(n_total, dim)
    return (e_full[:n_users], e_full[n_users:])


# speedup vs baseline: 1.3307x; 1.3307x over previous
import functools

import jax
import jax.numpy as jnp
from jax import lax
from jax.experimental import pallas as pl
from jax.experimental.pallas import tpu as pltpu
from jax.experimental.pallas import tpu_sc as plsc

_N_SC = 2
_N_SUB = 16
_CHUNK = 128
_HALF = 32
_GRPS = _CHUNK // 16


_UNROLL = 6
_RING = 3


def _layer_body(n_total, nch, ep, ibp, wp, zrows, out, acc, ib, wb, rows,
                contrib, *sems):
    c = lax.axis_index("c")
    s = lax.axis_index("s")
    rows_per_sub = n_total // _N_SUB
    roff = pl.multiple_of(s * rows_per_sub, 8)
    gsem = sems[0:3]
    ssem = sems[3:6]
    isem = sems[6:12]

    pltpu.sync_copy(zrows, acc.at[pl.ds(roff, rows_per_sub)])
    plsc.subcore_barrier()

    ep_c = ep.at[c]
    tile_base = s * nch

    def stage(j, slot):
        pltpu.async_copy(ibp.at[tile_base + j], ib.at[slot], isem[slot])
        pltpu.async_copy(wp.at[tile_base + j], wb.at[slot], isem[slot])

    def stage_wait(slot):
        pltpu.make_async_copy(ibp.at[tile_base], ib.at[slot],
                              isem[slot]).wait()
        pltpu.make_async_copy(wp.at[tile_base], wb.at[slot],
                              isem[slot]).wait()

    def gather(slot, b):
        pltpu.async_copy(ep_c.at[ib.at[slot, 0]], rows.at[b], gsem[b])

    def gather_wait(b):
        pltpu.make_async_copy(ep_c.at[pl.ds(0, _CHUNK)], rows.at[b],
                              gsem[b]).wait()

    def scat(slot, b):
        pltpu.async_copy(contrib.at[b], acc.at[ib.at[slot, 1]], ssem[b],
                         add=True)

    def scat_wait(b):
        pltpu.make_async_copy(contrib.at[b], acc.at[pl.ds(0, _CHUNK)],
                              ssem[b]).wait()

    def compute(b, slot):
        def group_body(g, carry):
            gb = pl.multiple_of(g * 16, 16)
            wgrp = wb[slot, pl.ds(gb, 16)]
            for e2 in range(16):
                e = gb + e2
                wv = wgrp[e2]
                contrib[b, e, pl.ds(0, 16)] = rows[b, e, pl.ds(0, 16)] * wv
                contrib[b, e, pl.ds(16, 16)] = rows[b, e, pl.ds(16, 16)] * wv
            return carry

        lax.fori_loop(0, _GRPS, group_body, 0)

    ni = nch // _UNROLL

    for j0 in range(3):
        stage(j0, j0)
    for j0 in range(2):
        stage_wait(j0)
        gather(j0, j0)

    def iter_body(i, carry):
        for k in range(_UNROLL):
            b = k % _RING
            j = _UNROLL * i + k
            gather_wait(b)
            if k >= 3:
                scat_wait(b)
            else:
                @pl.when(i > 0)
                def _():
                    scat_wait(b)
            if k < 3:
                stage(j + 3, (k + 3) % 6)
            else:
                @pl.when(i < ni - 1)
                def _():
                    stage(j + 3, (k + 3) % 6)
            if k < 4:
                stage_wait((k + 2) % 6)
                gather((k + 2) % 6, (k + 2) % _RING)
            else:
                @pl.when(i < ni - 1)
                def _():
                    stage_wait((k + 2) % 6)
                    gather((k + 2) % 6, (k + 2) % _RING)
            compute(b, k)
            scat(k, b)
        return carry

    lax.fori_loop(0, ni, iter_body, 0)
    for b in range(_RING):
        scat_wait(b)

    plsc.subcore_barrier()
    pltpu.sync_copy(acc.at[pl.ds(roff, rows_per_sub)],
                    out.at[c].at[pl.ds(roff, rows_per_sub)])


def _make_layer(n_total, nch):
    mesh = plsc.VectorSubcoreMesh(core_axis_name="c", subcore_axis_name="s")
    return pl.kernel(
        functools.partial(_layer_body, n_total, nch),
        out_type=jax.ShapeDtypeStruct((_N_SC, n_total, _HALF), jnp.float32),
        mesh=mesh,
        scratch_types=[
            pltpu.VMEM_SHARED((n_total, _HALF), jnp.float32),
            pltpu.VMEM((6, 2, _CHUNK), jnp.int32),
            pltpu.VMEM((6, _CHUNK), jnp.float32),
            pltpu.VMEM((_RING, _CHUNK, _HALF), jnp.float32),
            pltpu.VMEM((_RING, _CHUNK, _HALF), jnp.float32),
        ] + [pltpu.SemaphoreType.DMA] * 12,
        compiler_params=pltpu.CompilerParams(use_tc_tiling_on_sc=False,
                                             needs_layout_passes=False),
    )


def kernel(user_emb, item_emb, edge_weight, edge_index):
    n_users, dim = user_emb.shape
    n_total = n_users + item_emb.shape[0]
    n_edges = edge_weight.shape[0]
    n_layers = 3

    row_grain = _N_SUB * 8
    n_tpad = ((n_total + row_grain - 1) // row_grain) * row_grain

    grain = _N_SUB * _CHUNK * _UNROLL
    n_pad = ((n_edges + grain - 1) // grain) * grain
    pad = n_pad - n_edges
    nch = n_pad // (_N_SUB * _CHUNK)
    src = jnp.concatenate(
        [edge_index[1].astype(jnp.int32), jnp.zeros((pad,), jnp.int32)])
    dst = jnp.concatenate(
        [edge_index[0].astype(jnp.int32), jnp.zeros((pad,), jnp.int32)])
    w = jnp.concatenate(
        [edge_weight.astype(jnp.float32), jnp.zeros((pad,), jnp.float32)])
    ibp = jnp.stack(
        [src.reshape(-1, _CHUNK), dst.reshape(-1, _CHUNK)], axis=1)
    wp = w.reshape(-1, _CHUNK)

    e0 = jnp.concatenate([user_emb, item_emb], axis=0) \
        .reshape(n_total, _N_SC, _HALF).transpose(1, 0, 2)
    ep = jnp.zeros((_N_SC, n_tpad, _HALF), jnp.float32).at[:, :n_total].set(e0)
    zrows = jnp.zeros((n_tpad // _N_SUB, _HALF), jnp.float32)

    layer = _make_layer(n_tpad, nch)
    for _ in range(n_layers):
        ep = layer(ep, ibp, wp, zrows)

    e_full = ep[:, :n_total].transpose(1, 0, 2).reshape(n_total, dim)
    return (e_full[:n_users], e_full[n_users:])


# speedup vs baseline: 1.6310x; 1.2257x over previous
import functools

import jax
import jax.numpy as jnp
from jax import lax
from jax.experimental import pallas as pl
from jax.experimental.pallas import tpu as pltpu
from jax.experimental.pallas import tpu_sc as plsc

_N_SC = 2
_N_SUB = 16
_GCHUNK = 256
_SCHUNK = 128
_HALF = 32
_UNROLL = 4


def _layer_body(n_total, nch, ep, srcp, dstp, wp, zrows, out,
                acc, gidx, didx, wb, rows, contrib, *sems):
    c = lax.axis_index("c")
    s = lax.axis_index("s")
    rows_per_sub = n_total // _N_SUB
    roff = pl.multiple_of(s * rows_per_sub, 8)
    gsem = sems[0:2]
    ssem = sems[2:4]
    isem = sems[4:8]

    pltpu.sync_copy(zrows, acc.at[pl.ds(roff, rows_per_sub)])
    plsc.subcore_barrier()

    ep_c = ep.at[c]
    tile_base = s * nch

    def stage(j, slot):
        pltpu.async_copy(srcp.at[tile_base + j], gidx.at[slot], isem[slot])
        pltpu.async_copy(dstp.at[tile_base + j], didx.at[slot], isem[slot])
        pltpu.async_copy(wp.at[tile_base + j], wb.at[slot], isem[slot])

    def stage_wait(slot):
        pltpu.make_async_copy(srcp.at[tile_base], gidx.at[slot],
                              isem[slot]).wait()
        pltpu.make_async_copy(dstp.at[tile_base], didx.at[slot],
                              isem[slot]).wait()
        pltpu.make_async_copy(wp.at[tile_base], wb.at[slot],
                              isem[slot]).wait()

    def gather(slot, b):
        pltpu.async_copy(ep_c.at[gidx.at[slot]], rows.at[b], gsem[b])

    def gather_wait(b):
        pltpu.make_async_copy(ep_c.at[pl.ds(0, _GCHUNK)], rows.at[b],
                              gsem[b]).wait()

    def scat(slot, sub):
        pltpu.async_copy(contrib.at[sub], acc.at[didx.at[slot, sub]],
                         ssem[sub], add=True)

    def scat_wait(sub):
        pltpu.make_async_copy(contrib.at[sub], acc.at[pl.ds(0, _SCHUNK)],
                              ssem[sub]).wait()

    def compute(b, slot, sub):
        def group_body(g, carry):
            gb = pl.multiple_of(g * 16, 16)
            wgrp = wb[slot, pl.ds(sub * _SCHUNK + gb, 16)]
            for e2 in range(16):
                e = sub * _SCHUNK + gb + e2
                el = gb + e2
                wv = wgrp[e2]
                contrib[sub, el, pl.ds(0, 16)] = \
                    rows[b, e, pl.ds(0, 16)] * wv
                contrib[sub, el, pl.ds(16, 16)] = \
                    rows[b, e, pl.ds(16, 16)] * wv
            return carry

        lax.fori_loop(0, _SCHUNK // 16, group_body, 0)

    ni = nch // _UNROLL

    stage(0, 0)
    stage(1, 1)
    stage_wait(0)
    gather(0, 0)

    def iter_body(i, carry):
        for k in range(_UNROLL):
            b = k % 2
            j = _UNROLL * i + k
            gather_wait(b)
            if k >= 1:
                scat_wait(0)
                scat_wait(1)
            else:
                @pl.when(i > 0)
                def _():
                    scat_wait(0)
                    scat_wait(1)
            if k < 2:
                stage(j + 2, (k + 2) % 4)
            else:
                @pl.when(i < ni - 1)
                def _():
                    stage(j + 2, (k + 2) % 4)
            if k < 3:
                stage_wait((k + 1) % 4)
                gather((k + 1) % 4, 1 - b)
            else:
                @pl.when(i < ni - 1)
                def _():
                    stage_wait(0)
                    gather(0, 1 - b)
            for sub in range(2):
                compute(b, k, sub)
                scat(k, sub)
        return carry

    lax.fori_loop(0, ni, iter_body, 0)
    scat_wait(0)
    scat_wait(1)

    plsc.subcore_barrier()
    pltpu.sync_copy(acc.at[pl.ds(roff, rows_per_sub)],
                    out.at[c].at[pl.ds(roff, rows_per_sub)])


def _make_layer(n_total, nch):
    mesh = plsc.VectorSubcoreMesh(core_axis_name="c", subcore_axis_name="s")
    return pl.kernel(
        functools.partial(_layer_body, n_total, nch),
        out_type=jax.ShapeDtypeStruct((_N_SC, n_total, _HALF), jnp.float32),
        mesh=mesh,
        scratch_types=[
            pltpu.VMEM_SHARED((n_total, _HALF), jnp.float32),
            pltpu.VMEM((4, _GCHUNK), jnp.int32),
            pltpu.VMEM((4, 2, _SCHUNK), jnp.int32),
            pltpu.VMEM((4, _GCHUNK), jnp.float32),
            pltpu.VMEM((2, _GCHUNK, _HALF), jnp.float32),
            pltpu.VMEM((2, _SCHUNK, _HALF), jnp.float32),
        ] + [pltpu.SemaphoreType.DMA] * 8,
        compiler_params=pltpu.CompilerParams(use_tc_tiling_on_sc=False,
                                             needs_layout_passes=False),
    )


def kernel(user_emb, item_emb, edge_weight, edge_index):
    n_users, dim = user_emb.shape
    n_total = n_users + item_emb.shape[0]
    n_edges = edge_weight.shape[0]
    n_layers = 3

    row_grain = _N_SUB * 8
    n_tpad = ((n_total + row_grain - 1) // row_grain) * row_grain

    grain = _N_SUB * _GCHUNK * _UNROLL
    n_pad = ((n_edges + grain - 1) // grain) * grain
    pad = n_pad - n_edges
    nch = n_pad // (_N_SUB * _GCHUNK)
    src = jnp.concatenate(
        [edge_index[1].astype(jnp.int32), jnp.zeros((pad,), jnp.int32)])
    dst = jnp.concatenate(
        [edge_index[0].astype(jnp.int32), jnp.zeros((pad,), jnp.int32)])
    w = jnp.concatenate(
        [edge_weight.astype(jnp.float32), jnp.zeros((pad,), jnp.float32)])
    srcp = src.reshape(-1, _GCHUNK)
    dstp = dst.reshape(-1, 2, _SCHUNK)
    wp = w.reshape(-1, _GCHUNK)

    e0 = jnp.concatenate([user_emb, item_emb], axis=0) \
        .reshape(n_total, _N_SC, _HALF).transpose(1, 0, 2)
    ep = jnp.zeros((_N_SC, n_tpad, _HALF), jnp.float32).at[:, :n_total].set(e0)
    zrows = jnp.zeros((n_tpad // _N_SUB, _HALF), jnp.float32)

    layer = _make_layer(n_tpad, nch)
    for _ in range(n_layers):
        ep = layer(ep, srcp, dstp, wp, zrows)

    e_full = ep[:, :n_total].transpose(1, 0, 2).reshape(n_total, dim)
    return (e_full[:n_users], e_full[n_users:])


# speedup vs baseline: 1.7948x; 1.1004x over previous
import functools

import jax
import jax.numpy as jnp
from jax import lax
from jax.experimental import pallas as pl
from jax.experimental.pallas import tpu as pltpu
from jax.experimental.pallas import tpu_sc as plsc

_N_SC = 2
_N_SUB = 16
_GCHUNK = 256
_SCHUNK = 128
_HALF = 32
_UNROLL = 4


def _layer_body(n_total, nch, ep, srcp, dstp, wp, zrows, out,
                acc, gidx, didx, wb, rows, contrib, *sems):
    c = lax.axis_index("c")
    s = lax.axis_index("s")
    rows_per_sub = -(-n_total // (_N_SUB * 8)) * 8
    roff = pl.multiple_of(
        jnp.minimum(s * rows_per_sub, n_total - rows_per_sub), 8)
    gsem = sems[0:2]
    ssem = sems[2:4]
    isem = sems[4:8]

    ep_c = ep.at[c]
    tile_base = s * nch

    def stage(j, slot):
        pltpu.async_copy(srcp.at[tile_base + j], gidx.at[slot], isem[slot])
        pltpu.async_copy(dstp.at[tile_base + j], didx.at[slot], isem[slot])
        pltpu.async_copy(wp.at[tile_base + j], wb.at[slot], isem[slot])

    def stage_wait(slot):
        pltpu.make_async_copy(srcp.at[tile_base], gidx.at[slot],
                              isem[slot]).wait()
        pltpu.make_async_copy(dstp.at[tile_base], didx.at[slot],
                              isem[slot]).wait()
        pltpu.make_async_copy(wp.at[tile_base], wb.at[slot],
                              isem[slot]).wait()

    def gather(slot, b):
        pltpu.async_copy(ep_c.at[gidx.at[slot]], rows.at[b], gsem[b])

    def gather_wait(b):
        pltpu.make_async_copy(ep_c.at[pl.ds(0, _GCHUNK)], rows.at[b],
                              gsem[b]).wait()

    def scat(slot, sub):
        pltpu.async_copy(contrib.at[sub], acc.at[didx.at[slot, sub]],
                         ssem[sub], add=True)

    def scat_wait(sub):
        pltpu.make_async_copy(contrib.at[sub], acc.at[pl.ds(0, _SCHUNK)],
                              ssem[sub]).wait()

    def compute(b, slot, sub):
        def group_body(g, carry):
            gb = pl.multiple_of(g * 16, 16)
            wgrp = wb[slot, pl.ds(sub * _SCHUNK + gb, 16)]
            for e2 in range(16):
                e = sub * _SCHUNK + gb + e2
                el = gb + e2
                wv = wgrp[e2]
                contrib[sub, el, pl.ds(0, 16)] = \
                    rows[b, e, pl.ds(0, 16)] * wv
                contrib[sub, el, pl.ds(16, 16)] = \
                    rows[b, e, pl.ds(16, 16)] * wv
            return carry

        lax.fori_loop(0, _SCHUNK // 16, group_body, 0)

    ni = nch // _UNROLL

    stage(0, 0)
    stage(1, 1)
    stage_wait(0)
    gather(0, 0)

    pltpu.sync_copy(zrows, acc.at[pl.ds(roff, rows_per_sub)])
    plsc.subcore_barrier()

    def iter_body(i, carry):
        for k in range(_UNROLL):
            b = k % 2
            j = _UNROLL * i + k
            gather_wait(b)
            if k >= 1:
                scat_wait(0)
                scat_wait(1)
            else:
                @pl.when(i > 0)
                def _():
                    scat_wait(0)
                    scat_wait(1)
            if k < 2:
                stage(j + 2, (k + 2) % 4)
            else:
                @pl.when(i < ni - 1)
                def _():
                    stage(j + 2, (k + 2) % 4)
            if k < 3:
                stage_wait((k + 1) % 4)
                gather((k + 1) % 4, 1 - b)
            else:
                @pl.when(i < ni - 1)
                def _():
                    stage_wait(0)
                    gather(0, 1 - b)
            for sub in range(2):
                compute(b, k, sub)
                scat(k, sub)
        return carry

    lax.fori_loop(0, ni, iter_body, 0)
    scat_wait(0)
    scat_wait(1)

    plsc.subcore_barrier()
    pltpu.sync_copy(acc.at[pl.ds(roff, rows_per_sub)],
                    out.at[c].at[pl.ds(roff, rows_per_sub)])


def _make_layer(n_total, nch):
    mesh = plsc.VectorSubcoreMesh(core_axis_name="c", subcore_axis_name="s")
    return pl.kernel(
        functools.partial(_layer_body, n_total, nch),
        out_type=jax.ShapeDtypeStruct((_N_SC, n_total, _HALF), jnp.float32),
        mesh=mesh,
        scratch_types=[
            pltpu.VMEM_SHARED((n_total, _HALF), jnp.float32),
            pltpu.VMEM((4, _GCHUNK), jnp.int32),
            pltpu.VMEM((4, 2, _SCHUNK), jnp.int32),
            pltpu.VMEM((4, _GCHUNK), jnp.float32),
            pltpu.VMEM((2, _GCHUNK, _HALF), jnp.float32),
            pltpu.VMEM((2, _SCHUNK, _HALF), jnp.float32),
        ] + [pltpu.SemaphoreType.DMA] * 8,
        compiler_params=pltpu.CompilerParams(use_tc_tiling_on_sc=False,
                                             needs_layout_passes=False),
    )


def kernel(user_emb, item_emb, edge_weight, edge_index):
    n_users, dim = user_emb.shape
    n_total = n_users + item_emb.shape[0]
    n_edges = edge_weight.shape[0]
    n_layers = 3

    grain = _N_SUB * _GCHUNK * _UNROLL
    n_pad = ((n_edges + grain - 1) // grain) * grain
    pad = n_pad - n_edges
    nch = n_pad // (_N_SUB * _GCHUNK)
    src = jnp.concatenate(
        [edge_index[1].astype(jnp.int32), jnp.zeros((pad,), jnp.int32)])
    dst = jnp.concatenate(
        [edge_index[0].astype(jnp.int32), jnp.zeros((pad,), jnp.int32)])
    w = jnp.concatenate(
        [edge_weight.astype(jnp.float32), jnp.zeros((pad,), jnp.float32)])
    srcp = src.reshape(-1, _GCHUNK)
    dstp = dst.reshape(-1, 2, _SCHUNK)
    wp = w.reshape(-1, _GCHUNK)

    ep = jnp.concatenate([user_emb, item_emb], axis=0) \
        .reshape(n_total, _N_SC, _HALF).transpose(1, 0, 2)
    rows_per_sub = -(-n_total // (_N_SUB * 8)) * 8
    zrows = jnp.zeros((rows_per_sub, _HALF), jnp.float32)

    layer = _make_layer(n_total, nch)
    for _ in range(n_layers):
        ep = layer(ep, srcp, dstp, wp, zrows)

    e_full = ep.transpose(1, 0, 2).reshape(n_total, dim)
    return (e_full[:n_users], e_full[n_users:])


# speedup vs baseline: 1.9788x; 1.1025x over previous
import functools

import jax
import jax.numpy as jnp
from jax import lax
from jax.experimental import pallas as pl
from jax.experimental.pallas import tpu as pltpu
from jax.experimental.pallas import tpu_sc as plsc

_N_SC = 2
_N_SUB = 16
_GCHUNK = 256
_SCHUNK = 128
_HALF = 32
_UNROLL = 4


def _layer_body(n_total, nch, ep, srcp, dstp, wp, zrows, out,
                acc, gidx, didx, wb, rows, contrib, *sems):
    c = lax.axis_index("c")
    s = lax.axis_index("s")
    rows_per_sub = -(-n_total // (_N_SUB * 8)) * 8
    roff = pl.multiple_of(
        jnp.minimum(s * rows_per_sub, n_total - rows_per_sub), 8)
    gsem = sems[0:2]
    ssem = sems[2:4]
    isem = sems[4:8]

    ep_c = ep.at[c]
    tile_base = s * nch

    def stage(j, slot):
        pltpu.async_copy(srcp.at[tile_base + j], gidx.at[slot], isem[slot])
        pltpu.async_copy(dstp.at[tile_base + j], didx.at[slot], isem[slot])
        pltpu.async_copy(wp.at[tile_base + j], wb.at[slot], isem[slot])

    def stage_wait(slot):
        pltpu.make_async_copy(srcp.at[tile_base], gidx.at[slot],
                              isem[slot]).wait()
        pltpu.make_async_copy(dstp.at[tile_base], didx.at[slot],
                              isem[slot]).wait()
        pltpu.make_async_copy(wp.at[tile_base], wb.at[slot],
                              isem[slot]).wait()

    def gather(slot, b):
        pltpu.async_copy(ep_c.at[gidx.at[slot]], rows.at[b], gsem[b])

    def gather_wait(b):
        pltpu.make_async_copy(ep_c.at[pl.ds(0, _GCHUNK)], rows.at[b],
                              gsem[b]).wait()

    def scat(slot, sub):
        pltpu.async_copy(contrib.at[sub], acc.at[didx.at[slot, sub]],
                         ssem[sub], add=True)

    def scat_wait(sub):
        pltpu.make_async_copy(contrib.at[sub], acc.at[pl.ds(0, _SCHUNK)],
                              ssem[sub]).wait()

    def compute(b, slot, sub):
        def group_body(g, carry):
            gb = pl.multiple_of(g * 16, 16)
            wgrp = wb[slot, pl.ds(sub * _SCHUNK + gb, 16)]
            for e2 in range(16):
                e = sub * _SCHUNK + gb + e2
                el = gb + e2
                wv = wgrp[e2]
                contrib[sub, el, pl.ds(0, 16)] = \
                    rows[b, e, pl.ds(0, 16)] * wv
                contrib[sub, el, pl.ds(16, 16)] = \
                    rows[b, e, pl.ds(16, 16)] * wv
            return carry

        lax.fori_loop(0, _SCHUNK // 16, group_body, 0)

    ni = nch // _UNROLL

    stage(0, 0)
    stage(1, 1)
    stage_wait(0)
    gather(0, 0)

    pltpu.sync_copy(zrows, acc.at[pl.ds(roff, rows_per_sub)])
    plsc.subcore_barrier()

    def iter_body(i, carry):
        for k in range(_UNROLL):
            b = k % 2
            j = _UNROLL * i + k
            if k < 3:
                stage_wait((k + 1) % 4)
                gather((k + 1) % 4, 1 - b)
            else:
                @pl.when(i < ni - 1)
                def _():
                    stage_wait(0)
                    gather(0, 1 - b)
            gather_wait(b)
            if k >= 1:
                scat_wait(0)
                scat_wait(1)
            else:
                @pl.when(i > 0)
                def _():
                    scat_wait(0)
                    scat_wait(1)
            if k < 2:
                stage(j + 2, (k + 2) % 4)
            else:
                @pl.when(i < ni - 1)
                def _():
                    stage(j + 2, (k + 2) % 4)
            for sub in range(2):
                compute(b, k, sub)
                scat(k, sub)
        return carry

    lax.fori_loop(0, ni, iter_body, 0)
    scat_wait(0)
    scat_wait(1)

    plsc.subcore_barrier()
    pltpu.sync_copy(acc.at[pl.ds(roff, rows_per_sub)],
                    out.at[c].at[pl.ds(roff, rows_per_sub)])


def _make_layer(n_total, nch):
    mesh = plsc.VectorSubcoreMesh(core_axis_name="c", subcore_axis_name="s")
    return pl.kernel(
        functools.partial(_layer_body, n_total, nch),
        out_type=jax.ShapeDtypeStruct((_N_SC, n_total, _HALF), jnp.float32),
        mesh=mesh,
        scratch_types=[
            pltpu.VMEM_SHARED((n_total, _HALF), jnp.float32),
            pltpu.VMEM((4, _GCHUNK), jnp.int32),
            pltpu.VMEM((4, 2, _SCHUNK), jnp.int32),
            pltpu.VMEM((4, _GCHUNK), jnp.float32),
            pltpu.VMEM((2, _GCHUNK, _HALF), jnp.float32),
            pltpu.VMEM((2, _SCHUNK, _HALF), jnp.float32),
        ] + [pltpu.SemaphoreType.DMA] * 8,
        compiler_params=pltpu.CompilerParams(use_tc_tiling_on_sc=False,
                                             needs_layout_passes=False),
    )


def kernel(user_emb, item_emb, edge_weight, edge_index):
    n_users, dim = user_emb.shape
    n_total = n_users + item_emb.shape[0]
    n_edges = edge_weight.shape[0]
    n_layers = 3

    grain = _N_SUB * _GCHUNK * _UNROLL
    n_pad = ((n_edges + grain - 1) // grain) * grain
    pad = n_pad - n_edges
    nch = n_pad // (_N_SUB * _GCHUNK)
    src = jnp.concatenate(
        [edge_index[1].astype(jnp.int32), jnp.zeros((pad,), jnp.int32)])
    dst = jnp.concatenate(
        [edge_index[0].astype(jnp.int32), jnp.zeros((pad,), jnp.int32)])
    w = jnp.concatenate(
        [edge_weight.astype(jnp.float32), jnp.zeros((pad,), jnp.float32)])
    srcp = src.reshape(-1, _GCHUNK)
    dstp = dst.reshape(-1, 2, _SCHUNK)
    wp = w.reshape(-1, _GCHUNK)

    ep = jnp.concatenate([user_emb, item_emb], axis=0) \
        .reshape(n_total, _N_SC, _HALF).transpose(1, 0, 2)
    rows_per_sub = -(-n_total // (_N_SUB * 8)) * 8
    zrows = jnp.zeros((rows_per_sub, _HALF), jnp.float32)

    layer = _make_layer(n_total, nch)
    for _ in range(n_layers):
        ep = layer(ep, srcp, dstp, wp, zrows)

    e_full = ep.transpose(1, 0, 2).reshape(n_total, dim)
    return (e_full[:n_users], e_full[n_users:])
